# Initial kernel scaffold; baseline (speedup 1.0000x reference)
import jax
import jax.numpy as jnp
from jax.experimental import pallas as pl


def kernel(user_emb, item_emb, item_bias, edge_vals, edge_index, user_id, hist_item_ids, pos_target, replace_candidates, sampled_neg_targets, sample_idices):
    raise NotImplementedError("write your pallas kernel here")


# speedup vs baseline: 4.0209x; 4.0209x over previous
import functools

import jax
import jax.numpy as jnp
from jax import lax
from jax.experimental import pallas as pl
from jax.experimental.pallas import tpu as pltpu
from jax.experimental.pallas import tpu_sc as plsc

NI = 10000
NIP = 10240
ND = 128
NE = 320000
NB = 1024
NL = 50
NC, NS = 2, 16
NW = NC * NS
EK = 128
ECHUNKS = 80
EPT = EK * ECHUNKS
E_PAD = NW * EPT
RPT = NIP // NS
LP = 56
BPT = NB // NW


def _lane_sum(v, tmp_v):
  for sh in (8, 4, 2, 1):
    tmp_v[pl.ds(0, 16)] = v
    tmp_v[pl.ds(16, 16)] = v
    v = v + tmp_v[pl.ds(sh, 16)]
  return v


def _make_layer_kernel():
  mesh = plsc.VectorSubcoreMesh(core_axis_name="c", subcore_axis_name="s",
                                num_cores=NC, num_subcores=NS)

  @functools.partial(
      pl.kernel,
      out_type=jax.ShapeDtypeStruct((NC, NIP, ND), jnp.float32),
      mesh=mesh,
      compiler_params=pltpu.CompilerParams(needs_layout_passes=False),
      scratch_types=[
          pltpu.VMEM((EK,), jnp.int32),
          pltpu.VMEM((EK,), jnp.int32),
          pltpu.VMEM((EK,), jnp.float32),
          pltpu.VMEM((EK, ND), jnp.float32),
          pltpu.VMEM((128, ND), jnp.float32),
          pltpu.VMEM_SHARED((NIP, ND), jnp.float32),
          pltpu.SemaphoreType.DMA,
      ],
  )
  def layer(emb_hbm, src_hbm, dst_hbm, vals_hbm, out_hbm,
            sidx_v, didx_v, vals_v, rows_v, zbuf_v, acc_sh, sem):
    cid = lax.axis_index("c")
    sid = lax.axis_index("s")
    zero16 = jnp.zeros((16,), jnp.float32)

    @pl.loop(0, 128)
    def _zrow(i):
      for q in range(8):
        zbuf_v[i, pl.ds(q * 16, 16)] = zero16

    @pl.loop(0, 5)
    def _zacc(k):
      pltpu.sync_copy(zbuf_v, acc_sh.at[pl.ds(sid * RPT + k * 128, 128)])

    plsc.subcore_barrier()

    base = (cid * NS + sid) * EPT

    @pl.loop(0, ECHUNKS)
    def _chunk(j):
      off = base + j * EK
      pltpu.sync_copy(src_hbm.at[pl.ds(off, EK)], sidx_v)
      pltpu.sync_copy(dst_hbm.at[pl.ds(off, EK)], didx_v)
      pltpu.sync_copy(vals_hbm.at[pl.ds(off, EK)], vals_v)
      pltpu.async_copy(emb_hbm.at[sidx_v], rows_v, sem).wait()

      @pl.loop(0, EK // 16)
      def _scale(g):
        v16 = vals_v[pl.ds(g * 16, 16)]
        for e in range(16):
          s = jnp.broadcast_to(v16[e], (16,))
          row = g * 16 + e
          for q in range(8):
            sl = pl.ds(q * 16, 16)
            rows_v[row, sl] = rows_v[row, sl] * s

      pltpu.sync_copy(rows_v, acc_sh.at[didx_v], add=True)

    plsc.subcore_barrier()

    @pl.loop(0, 5)
    def _wout(k):
      r0 = sid * RPT + k * 128
      pltpu.sync_copy(acc_sh.at[pl.ds(r0, 128)], out_hbm.at[cid, pl.ds(r0, 128)])

  return layer


def _make_score_kernel():
  mesh = plsc.VectorSubcoreMesh(core_axis_name="c", subcore_axis_name="s",
                                num_cores=NC, num_subcores=NS)

  @functools.partial(
      pl.kernel,
      out_type=(
          jax.ShapeDtypeStruct((NB,), jnp.float32),
          jax.ShapeDtypeStruct((NB,), jnp.float32),
          jax.ShapeDtypeStruct((NB,), jnp.float32),
      ),
      mesh=mesh,
      compiler_params=pltpu.CompilerParams(needs_layout_passes=False),
      scratch_types=[
          pltpu.VMEM((BPT,), jnp.int32),
          pltpu.VMEM((BPT,), jnp.int32),
          pltpu.VMEM((BPT,), jnp.int32),
          pltpu.VMEM((BPT,), jnp.int32),
          pltpu.VMEM((BPT, LP), jnp.int32),
          pltpu.VMEM((BPT, ND), jnp.float32),
          pltpu.VMEM((BPT, ND), jnp.float32),
          pltpu.VMEM((BPT, ND), jnp.float32),
          pltpu.VMEM((BPT, ND), jnp.float32),
          pltpu.VMEM((LP, ND), jnp.float32),
          pltpu.VMEM((NI,), jnp.float32),
          pltpu.VMEM((BPT,), jnp.float32),
          pltpu.VMEM((BPT,), jnp.float32),
          pltpu.VMEM((BPT,), jnp.float32),
          pltpu.VMEM((32,), jnp.float32),
          pltpu.SemaphoreType.DMA,
      ],
  )
  def score(items_hbm, user_hbm, bias_hbm, uid_hbm, hist_hbm, pos_hbm,
            neg0_hbm, neg1_hbm, p_hbm, n0_hbm, n1_hbm,
            uidx_v, pidx_v, n0idx_v, n1idx_v, hidx_v,
            urows_v, prows_v, n0rows_v, n1rows_v, hrows_v,
            bias_v, pbuf_v, n0buf_v, n1buf_v, rot_v, sem):
    cid = lax.axis_index("c")
    sid = lax.axis_index("s")
    bb = (cid * NS + sid) * BPT

    pltpu.sync_copy(uid_hbm.at[pl.ds(bb, BPT)], uidx_v)
    pltpu.sync_copy(pos_hbm.at[pl.ds(bb, BPT)], pidx_v)
    pltpu.sync_copy(neg0_hbm.at[pl.ds(bb, BPT)], n0idx_v)
    pltpu.sync_copy(neg1_hbm.at[pl.ds(bb, BPT)], n1idx_v)
    pltpu.sync_copy(hist_hbm.at[pl.ds(bb, BPT)], hidx_v)
    pltpu.sync_copy(bias_hbm, bias_v)
    pltpu.async_copy(user_hbm.at[uidx_v], urows_v, sem).wait()
    pltpu.async_copy(items_hbm.at[pidx_v], prows_v, sem).wait()
    pltpu.async_copy(items_hbm.at[n0idx_v], n0rows_v, sem).wait()
    pltpu.async_copy(items_hbm.at[n1idx_v], n1rows_v, sem).wait()

    zero16 = jnp.zeros((16,), jnp.float32)
    lanes = lax.iota(jnp.int32, 16)
    inv_l = jnp.float32(1.0 / NL)

    @pl.loop(0, BPT)
    def _b(b):
      pltpu.async_copy(items_hbm.at[hidx_v.at[b]], hrows_v, sem).wait()

      def hbody(r, accs):
        return tuple(a + hrows_v[r, pl.ds(q * 16, 16)]
                     for q, a in enumerate(accs))

      accs = lax.fori_loop(0, NL, hbody, (zero16,) * 8)

      pd = zero16
      n0d = zero16
      n1d = zero16
      for q in range(8):
        sl = pl.ds(q * 16, 16)
        ctx = urows_v[b, sl] + accs[q] * inv_l
        pd = pd + ctx * prows_v[b, sl]
        n0d = n0d + ctx * n0rows_v[b, sl]
        n1d = n1d + ctx * n1rows_v[b, sl]
      grp = (b // 16) * 16
      sel = lanes == (b - grp)
      gsl = pl.ds(grp, 16)
      pbuf_v[gsl] = jnp.where(sel, _lane_sum(pd, rot_v), pbuf_v[gsl])
      n0buf_v[gsl] = jnp.where(sel, _lane_sum(n0d, rot_v), n0buf_v[gsl])
      n1buf_v[gsl] = jnp.where(sel, _lane_sum(n1d, rot_v), n1buf_v[gsl])

    for g in range(BPT // 16):
      gsl = pl.ds(g * 16, 16)
      pbuf_v[gsl] = pbuf_v[gsl] + plsc.load_gather(bias_v, [pidx_v[gsl]])
      n0buf_v[gsl] = n0buf_v[gsl] + plsc.load_gather(bias_v, [n0idx_v[gsl]])
      n1buf_v[gsl] = n1buf_v[gsl] + plsc.load_gather(bias_v, [n1idx_v[gsl]])

    pltpu.sync_copy(pbuf_v, p_hbm.at[pl.ds(bb, BPT)])
    pltpu.sync_copy(n0buf_v, n0_hbm.at[pl.ds(bb, BPT)])
    pltpu.sync_copy(n1buf_v, n1_hbm.at[pl.ds(bb, BPT)])

  return score


def _combine_partials(p):
  def body(p_ref, o_ref):
    o_ref[...] = p_ref[0] + p_ref[1]

  return pl.pallas_call(
      body,
      grid=(10,),
      in_specs=[pl.BlockSpec((2, NIP // 10, ND), lambda i: (0, i, 0))],
      out_specs=pl.BlockSpec((NIP // 10, ND), lambda i: (i, 0)),
      out_shape=jax.ShapeDtypeStruct((NIP, ND), jnp.float32),
  )(p)


def _mean_items(emb0, emb1, p1):
  def body(e0_ref, e1_ref, p_ref, o_ref):
    o_ref[...] = (e0_ref[...] + e1_ref[...] + p_ref[0] + p_ref[1]) * (1.0 / 3.0)

  r = NIP // 10
  return pl.pallas_call(
      body,
      grid=(10,),
      in_specs=[
          pl.BlockSpec((r, ND), lambda i: (i, 0)),
          pl.BlockSpec((r, ND), lambda i: (i, 0)),
          pl.BlockSpec((2, r, ND), lambda i: (0, i, 0)),
      ],
      out_specs=pl.BlockSpec((r, ND), lambda i: (i, 0)),
      out_shape=jax.ShapeDtypeStruct((NIP, ND), jnp.float32),
  )(emb0, emb1, p1)


def _loss_tc(p, n0, n1):
  def body(p_ref, n0_ref, n1_ref, o_ref):
    ps = p_ref[...]

    def sp(z):
      return jnp.maximum(z, 0.0) + jnp.log1p(jnp.exp(-jnp.abs(z)))

    o_ref[...] = sp(n0_ref[...] - ps) + sp(n1_ref[...] - ps)

  return pl.pallas_call(
      body,
      out_shape=jax.ShapeDtypeStruct((8, 128), jnp.float32),
  )(p.reshape(8, 128), n0.reshape(8, 128), n1.reshape(8, 128))


def kernel(user_emb, item_emb, item_bias, edge_vals, edge_index, user_id,
           hist_item_ids, pos_target, replace_candidates, sampled_neg_targets,
           sample_idices):
  del replace_candidates

  pad = E_PAD - NE
  idt = edge_index.dtype
  dst = jnp.concatenate([edge_index[0], jnp.zeros((pad,), idt)]).astype(jnp.int32)
  src = jnp.concatenate([edge_index[1], jnp.zeros((pad,), idt)]).astype(jnp.int32)
  vals = jnp.concatenate([edge_vals, jnp.zeros((pad,), jnp.float32)])

  item_emb_pad = jnp.pad(item_emb, ((0, NIP - NI), (0, 0)))
  layer = _make_layer_kernel()
  p0 = layer(item_emb_pad, src, dst, vals)
  emb1 = _combine_partials(p0)
  p1 = layer(emb1, src, dst, vals)
  items = _mean_items(item_emb_pad, emb1, p1)

  score = _make_score_kernel()
  hist_pad = jnp.pad(hist_item_ids.astype(jnp.int32), ((0, 0), (0, LP - NL)))
  p, n0, n1 = score(
      items, user_emb, item_bias.reshape(NI),
      user_id.astype(jnp.int32), hist_pad,
      pos_target.astype(jnp.int32),
      sampled_neg_targets[:, 0].astype(jnp.int32),
      sampled_neg_targets[:, 1].astype(jnp.int32),
  )

  loss = _loss_tc(p, n0, n1).reshape(NB, 1)
  return (loss, sample_idices)


# speedup vs baseline: 5.0871x; 1.2652x over previous
import functools

import jax
import jax.numpy as jnp
from jax import lax
from jax.experimental import pallas as pl
from jax.experimental.pallas import tpu as pltpu
from jax.experimental.pallas import tpu_sc as plsc

NI = 10000
NIP = 10240
ND = 128
NE = 320000
NB = 1024
NL = 50
NC, NS = 2, 16
NW = NC * NS
EK = 128
ECHUNKS = 80
SUP = 8
NSUP = ECHUNKS // SUP
EPT = EK * ECHUNKS
E_PAD = NW * EPT
RPT = NIP // NS
LP = 56
BPT = NB // NW


def _lane_sum(v, tmp_v):
  for sh in (8, 4, 2, 1):
    tmp_v[pl.ds(0, 16)] = v
    tmp_v[pl.ds(16, 16)] = v
    v = v + tmp_v[pl.ds(sh, 16)]
  return v


def _make_layer_kernel():
  mesh = plsc.VectorSubcoreMesh(core_axis_name="c", subcore_axis_name="s",
                                num_cores=NC, num_subcores=NS)

  @functools.partial(
      pl.kernel,
      out_type=jax.ShapeDtypeStruct((NC, NIP, ND), jnp.float32),
      mesh=mesh,
      compiler_params=pltpu.CompilerParams(needs_layout_passes=False),
      scratch_types=[
          pltpu.VMEM((2, SUP, EK), jnp.int32),
          pltpu.VMEM((2, SUP, EK), jnp.int32),
          pltpu.VMEM((2, SUP, EK), jnp.float32),
          pltpu.VMEM((EK, ND), jnp.float32),
          pltpu.VMEM((EK, ND), jnp.float32),
          pltpu.VMEM_SHARED((NIP, ND), jnp.float32),
          pltpu.SemaphoreType.DMA,
          pltpu.SemaphoreType.DMA,
      ],
  )
  def layer(emb_hbm, src_hbm, dst_hbm, vals_hbm, out_hbm,
            sidx_v, didx_v, vals_v, rows0_v, rows1_v, acc_sh, sem, ssem):
    cid = lax.axis_index("c")
    sid = lax.axis_index("s")
    tid = cid * NS + sid
    zero16 = jnp.zeros((16,), jnp.float32)

    def stage_fire(sp, sb):
      sl = pl.ds(sp * SUP, SUP)
      pltpu.async_copy(src_hbm.at[tid, sl], sidx_v.at[sb], ssem)
      pltpu.async_copy(dst_hbm.at[tid, sl], didx_v.at[sb], ssem)
      pltpu.async_copy(vals_hbm.at[tid, sl], vals_v.at[sb], ssem)

    def stage_drain(sp, sb):
      sl = pl.ds(sp * SUP, SUP)
      pltpu.make_async_copy(src_hbm.at[tid, sl], sidx_v.at[sb], ssem).wait()
      pltpu.make_async_copy(dst_hbm.at[tid, sl], didx_v.at[sb], ssem).wait()
      pltpu.make_async_copy(vals_hbm.at[tid, sl], vals_v.at[sb], ssem).wait()

    def fire(sb, c, buf):
      pltpu.async_copy(emb_hbm.at[sidx_v.at[sb, c]], buf, sem)

    def drain(sb, c, buf):
      pltpu.make_async_copy(emb_hbm.at[sidx_v.at[sb, c]], buf, sem).wait()

    def process(sb, c, buf):
      @pl.loop(0, EK // 16)
      def _scale(g):
        v16 = vals_v[sb, c, pl.ds(g * 16, 16)]
        for e in range(16):
          svec = jnp.broadcast_to(v16[e], (16,))
          row = g * 16 + e
          for q in range(8):
            qsl = pl.ds(q * 16, 16)
            buf[row, qsl] = buf[row, qsl] * svec

      pltpu.sync_copy(buf, acc_sh.at[didx_v.at[sb, c]], add=True)

    stage_fire(0, 0)

    @pl.loop(0, EK)
    def _zrow(i):
      for q in range(8):
        rows1_v[i, pl.ds(q * 16, 16)] = zero16

    @pl.loop(0, RPT // EK)
    def _zacc(k):
      pltpu.sync_copy(rows1_v, acc_sh.at[pl.ds(sid * RPT + k * EK, EK)])

    plsc.subcore_barrier()

    @pl.loop(0, NSUP)
    def _sup(sp):
      sb = sp % 2
      stage_drain(sp, sb)

      @pl.when(sp + 1 < NSUP)
      def _nextstage():
        stage_fire(sp + 1, 1 - sb)

      fire(sb, 0, rows0_v)

      @pl.loop(0, SUP // 2)
      def _c2(c2):
        c0 = 2 * c2
        c1 = c0 + 1
        drain(sb, c0, rows0_v)
        fire(sb, c1, rows1_v)
        process(sb, c0, rows0_v)
        drain(sb, c1, rows1_v)

        @pl.when(c1 + 1 < SUP)
        def _nextrow():
          fire(sb, c1 + 1, rows0_v)

        process(sb, c1, rows1_v)

    plsc.subcore_barrier()

    @pl.loop(0, RPT // EK)
    def _wout(k):
      r0 = sid * RPT + k * EK
      pltpu.sync_copy(acc_sh.at[pl.ds(r0, EK)], out_hbm.at[cid, pl.ds(r0, EK)])

  return layer


def _make_score_kernel():
  mesh = plsc.VectorSubcoreMesh(core_axis_name="c", subcore_axis_name="s",
                                num_cores=NC, num_subcores=NS)

  @functools.partial(
      pl.kernel,
      out_type=(
          jax.ShapeDtypeStruct((NB,), jnp.float32),
          jax.ShapeDtypeStruct((NB,), jnp.float32),
          jax.ShapeDtypeStruct((NB,), jnp.float32),
      ),
      mesh=mesh,
      compiler_params=pltpu.CompilerParams(needs_layout_passes=False),
      scratch_types=[
          pltpu.VMEM((BPT,), jnp.int32),
          pltpu.VMEM((BPT,), jnp.int32),
          pltpu.VMEM((BPT,), jnp.int32),
          pltpu.VMEM((BPT,), jnp.int32),
          pltpu.VMEM((BPT, LP), jnp.int32),
          pltpu.VMEM((BPT, ND), jnp.float32),
          pltpu.VMEM((BPT, ND), jnp.float32),
          pltpu.VMEM((BPT, ND), jnp.float32),
          pltpu.VMEM((BPT, ND), jnp.float32),
          pltpu.VMEM((LP, ND), jnp.float32),
          pltpu.VMEM((LP, ND), jnp.float32),
          pltpu.VMEM((NI,), jnp.float32),
          pltpu.VMEM((BPT,), jnp.float32),
          pltpu.VMEM((BPT,), jnp.float32),
          pltpu.VMEM((BPT,), jnp.float32),
          pltpu.VMEM((32,), jnp.float32),
          pltpu.SemaphoreType.DMA,
          pltpu.SemaphoreType.DMA,
      ],
  )
  def score(items_hbm, user_hbm, bias_hbm, uid_hbm, hist_hbm, pos_hbm,
            neg0_hbm, neg1_hbm, p_hbm, n0_hbm, n1_hbm,
            uidx_v, pidx_v, n0idx_v, n1idx_v, hidx_v,
            urows_v, prows_v, n0rows_v, n1rows_v, hrows0_v, hrows1_v,
            bias_v, pbuf_v, n0buf_v, n1buf_v, rot_v, bsem, hsem):
    cid = lax.axis_index("c")
    sid = lax.axis_index("s")
    bb = (cid * NS + sid) * BPT

    pltpu.sync_copy(uid_hbm.at[pl.ds(bb, BPT)], uidx_v)
    pltpu.sync_copy(pos_hbm.at[pl.ds(bb, BPT)], pidx_v)
    pltpu.sync_copy(neg0_hbm.at[pl.ds(bb, BPT)], n0idx_v)
    pltpu.sync_copy(neg1_hbm.at[pl.ds(bb, BPT)], n1idx_v)
    pltpu.sync_copy(hist_hbm.at[pl.ds(bb, BPT)], hidx_v)

    def hfire(b, buf):
      pltpu.async_copy(items_hbm.at[hidx_v.at[b]], buf, hsem)

    def hdrain(b, buf):
      pltpu.make_async_copy(items_hbm.at[hidx_v.at[b]], buf, hsem).wait()

    pltpu.async_copy(user_hbm.at[uidx_v], urows_v, bsem)
    pltpu.async_copy(items_hbm.at[pidx_v], prows_v, bsem)
    pltpu.async_copy(items_hbm.at[n0idx_v], n0rows_v, bsem)
    pltpu.async_copy(items_hbm.at[n1idx_v], n1rows_v, bsem)
    hfire(0, hrows0_v)
    pltpu.sync_copy(bias_hbm, bias_v)
    pltpu.make_async_copy(user_hbm.at[uidx_v], urows_v, bsem).wait()
    pltpu.make_async_copy(items_hbm.at[pidx_v], prows_v, bsem).wait()
    pltpu.make_async_copy(items_hbm.at[n0idx_v], n0rows_v, bsem).wait()
    pltpu.make_async_copy(items_hbm.at[n1idx_v], n1rows_v, bsem).wait()

    zero16 = jnp.zeros((16,), jnp.float32)
    lanes = lax.iota(jnp.int32, 16)
    inv_l = jnp.float32(1.0 / NL)

    def compute(b, hbuf):
      def hbody(r, accs):
        return tuple(a + hbuf[r, pl.ds(q * 16, 16)]
                     for q, a in enumerate(accs))

      accs = lax.fori_loop(0, NL, hbody, (zero16,) * 8)

      pd = zero16
      n0d = zero16
      n1d = zero16
      for q in range(8):
        sl = pl.ds(q * 16, 16)
        ctx = urows_v[b, sl] + accs[q] * inv_l
        pd = pd + ctx * prows_v[b, sl]
        n0d = n0d + ctx * n0rows_v[b, sl]
        n1d = n1d + ctx * n1rows_v[b, sl]
      grp = (b // 16) * 16
      sel = lanes == (b - grp)
      gsl = pl.ds(grp, 16)
      pbuf_v[gsl] = jnp.where(sel, _lane_sum(pd, rot_v), pbuf_v[gsl])
      n0buf_v[gsl] = jnp.where(sel, _lane_sum(n0d, rot_v), n0buf_v[gsl])
      n1buf_v[gsl] = jnp.where(sel, _lane_sum(n1d, rot_v), n1buf_v[gsl])

    @pl.loop(0, BPT // 2)
    def _b2(b2):
      b0 = 2 * b2
      b1 = b0 + 1
      hdrain(b0, hrows0_v)
      hfire(b1, hrows1_v)
      compute(b0, hrows0_v)
      hdrain(b1, hrows1_v)

      @pl.when(b2 + 1 < BPT // 2)
      def _next():
        hfire(b1 + 1, hrows0_v)

      compute(b1, hrows1_v)

    for g in range(BPT // 16):
      gsl = pl.ds(g * 16, 16)
      pbuf_v[gsl] = pbuf_v[gsl] + plsc.load_gather(bias_v, [pidx_v[gsl]])
      n0buf_v[gsl] = n0buf_v[gsl] + plsc.load_gather(bias_v, [n0idx_v[gsl]])
      n1buf_v[gsl] = n1buf_v[gsl] + plsc.load_gather(bias_v, [n1idx_v[gsl]])

    pltpu.sync_copy(pbuf_v, p_hbm.at[pl.ds(bb, BPT)])
    pltpu.sync_copy(n0buf_v, n0_hbm.at[pl.ds(bb, BPT)])
    pltpu.sync_copy(n1buf_v, n1_hbm.at[pl.ds(bb, BPT)])

  return score


def _combine_partials(p):
  def body(p_ref, o_ref):
    o_ref[...] = p_ref[0] + p_ref[1]

  return pl.pallas_call(
      body,
      grid=(10,),
      in_specs=[pl.BlockSpec((2, NIP // 10, ND), lambda i: (0, i, 0))],
      out_specs=pl.BlockSpec((NIP // 10, ND), lambda i: (i, 0)),
      out_shape=jax.ShapeDtypeStruct((NIP, ND), jnp.float32),
  )(p)


def _mean_items(emb0, emb1, p1):
  def body(e0_ref, e1_ref, p_ref, o_ref):
    o_ref[...] = (e0_ref[...] + e1_ref[...] + p_ref[0] + p_ref[1]) * (1.0 / 3.0)

  r = NIP // 10
  return pl.pallas_call(
      body,
      grid=(10,),
      in_specs=[
          pl.BlockSpec((r, ND), lambda i: (i, 0)),
          pl.BlockSpec((r, ND), lambda i: (i, 0)),
          pl.BlockSpec((2, r, ND), lambda i: (0, i, 0)),
      ],
      out_specs=pl.BlockSpec((r, ND), lambda i: (i, 0)),
      out_shape=jax.ShapeDtypeStruct((NIP, ND), jnp.float32),
  )(emb0, emb1, p1)


def _loss_tc(p, n0, n1):
  def body(p_ref, n0_ref, n1_ref, o_ref):
    ps = p_ref[...]

    def sp(z):
      return jnp.maximum(z, 0.0) + jnp.log1p(jnp.exp(-jnp.abs(z)))

    o_ref[...] = sp(n0_ref[...] - ps) + sp(n1_ref[...] - ps)

  return pl.pallas_call(
      body,
      out_shape=jax.ShapeDtypeStruct((8, 128), jnp.float32),
  )(p.reshape(8, 128), n0.reshape(8, 128), n1.reshape(8, 128))


def kernel(user_emb, item_emb, item_bias, edge_vals, edge_index, user_id,
           hist_item_ids, pos_target, replace_candidates, sampled_neg_targets,
           sample_idices):
  del replace_candidates

  pad = E_PAD - NE
  idt = edge_index.dtype
  dst = jnp.concatenate([edge_index[0], jnp.zeros((pad,), idt)])
  dst = dst.astype(jnp.int32).reshape(NW, ECHUNKS, EK)
  src = jnp.concatenate([edge_index[1], jnp.zeros((pad,), idt)])
  src = src.astype(jnp.int32).reshape(NW, ECHUNKS, EK)
  vals = jnp.concatenate([edge_vals, jnp.zeros((pad,), jnp.float32)])
  vals = vals.reshape(NW, ECHUNKS, EK)

  item_emb_pad = jnp.pad(item_emb, ((0, NIP - NI), (0, 0)))
  layer = _make_layer_kernel()
  p0 = layer(item_emb_pad, src, dst, vals)
  emb1 = _combine_partials(p0)
  p1 = layer(emb1, src, dst, vals)
  items = _mean_items(item_emb_pad, emb1, p1)

  score = _make_score_kernel()
  hist_pad = jnp.pad(hist_item_ids.astype(jnp.int32), ((0, 0), (0, LP - NL)))
  p, n0, n1 = score(
      items, user_emb, item_bias.reshape(NI),
      user_id.astype(jnp.int32), hist_pad,
      pos_target.astype(jnp.int32),
      sampled_neg_targets[:, 0].astype(jnp.int32),
      sampled_neg_targets[:, 1].astype(jnp.int32),
  )

  loss = _loss_tc(p, n0, n1).reshape(NB, 1)
  return (loss, sample_idices)


# speedup vs baseline: 6.1128x; 1.2016x over previous
import functools

import jax
import jax.numpy as jnp
from jax import lax
from jax.experimental import pallas as pl
from jax.experimental.pallas import tpu as pltpu
from jax.experimental.pallas import tpu_sc as plsc

NI = 10000
NIP = 10240
ND = 128
NE = 320000
NB = 1024
NL = 50
NC, NS = 2, 16
NW = NC * NS
EK = 128
ECHUNKS = 80
SUP = 8
NSUP = ECHUNKS // SUP
NSUP0 = 17
NSUP1 = 2 * NSUP - NSUP0
ESUP = SUP * EK
EPT = EK * ECHUNKS
E_PAD = NW * EPT
TOTSUP = E_PAD // ESUP
RPT = NIP // NS
LP = 56
BPT = NB // NW


def _make_layer_kernel():
  mesh = plsc.VectorSubcoreMesh(core_axis_name="c", subcore_axis_name="s",
                                num_cores=NC, num_subcores=NS)

  @functools.partial(
      pl.kernel,
      out_type=jax.ShapeDtypeStruct((NC, NIP, ND), jnp.float32),
      mesh=mesh,
      compiler_params=pltpu.CompilerParams(needs_layout_passes=False),
      scratch_types=[
          pltpu.VMEM((2, SUP, EK), jnp.int32),
          pltpu.VMEM((2, SUP, EK), jnp.int32),
          pltpu.VMEM((2, SUP, EK), jnp.float32),
          pltpu.VMEM((EK, ND), jnp.float32),
          pltpu.VMEM((EK, ND), jnp.float32),
          pltpu.VMEM_SHARED((NIP, ND), jnp.float32),
          pltpu.SemaphoreType.DMA,
          pltpu.SemaphoreType.DMA,
      ],
  )
  def layer(emb_hbm, src_hbm, dst_hbm, vals_hbm, out_hbm,
            sidx_v, didx_v, vals_v, rows0_v, rows1_v, acc_sh, sem, ssem):
    cid = lax.axis_index("c")
    sid = lax.axis_index("s")
    zero16 = jnp.zeros((16,), jnp.float32)
    nsup = jnp.where(cid == 0, NSUP0, NSUP1)
    bsu = jnp.where(cid == 0, sid * NSUP0, NS * NSUP0 + sid * NSUP1)
    bsu = jnp.minimum(bsu, TOTSUP - 1)

    def stage_fire(sp, sb):
      pltpu.async_copy(src_hbm.at[bsu + sp], sidx_v.at[sb], ssem)
      pltpu.async_copy(dst_hbm.at[bsu + sp], didx_v.at[sb], ssem)
      pltpu.async_copy(vals_hbm.at[bsu + sp], vals_v.at[sb], ssem)

    def stage_drain(sp, sb):
      pltpu.make_async_copy(src_hbm.at[bsu + sp], sidx_v.at[sb], ssem).wait()
      pltpu.make_async_copy(dst_hbm.at[bsu + sp], didx_v.at[sb], ssem).wait()
      pltpu.make_async_copy(vals_hbm.at[bsu + sp], vals_v.at[sb], ssem).wait()

    def fire(sb, c, buf):
      pltpu.async_copy(emb_hbm.at[sidx_v.at[sb, c]], buf, sem)

    def drain(sb, c, buf):
      pltpu.make_async_copy(emb_hbm.at[sidx_v.at[sb, c]], buf, sem).wait()

    def process(sb, c, buf):
      @pl.loop(0, EK // 16)
      def _scale(g):
        v16 = vals_v[sb, c, pl.ds(g * 16, 16)]
        for e in range(16):
          svec = jnp.broadcast_to(v16[e], (16,))
          row = g * 16 + e
          for q in range(8):
            qsl = pl.ds(q * 16, 16)
            buf[row, qsl] = buf[row, qsl] * svec

      pltpu.sync_copy(buf, acc_sh.at[didx_v.at[sb, c]], add=True)

    @pl.when(nsup > 0)
    def _prologue():
      stage_fire(0, 0)

    @pl.loop(0, EK)
    def _zrow(i):
      for q in range(8):
        rows1_v[i, pl.ds(q * 16, 16)] = zero16

    @pl.loop(0, RPT // EK)
    def _zacc(k):
      pltpu.sync_copy(rows1_v, acc_sh.at[pl.ds(sid * RPT + k * EK, EK)])

    plsc.subcore_barrier()

    @pl.loop(0, nsup)
    def _sup(sp):
      sb = sp % 2
      stage_drain(sp, sb)

      @pl.when(sp + 1 < nsup)
      def _nextstage():
        stage_fire(sp + 1, 1 - sb)

      fire(sb, 0, rows0_v)

      @pl.loop(0, SUP // 2)
      def _c2(c2):
        c0 = 2 * c2
        c1 = c0 + 1
        drain(sb, c0, rows0_v)
        fire(sb, c1, rows1_v)
        process(sb, c0, rows0_v)
        drain(sb, c1, rows1_v)

        @pl.when(c1 + 1 < SUP)
        def _nextrow():
          fire(sb, c1 + 1, rows0_v)

        process(sb, c1, rows1_v)

    plsc.subcore_barrier()

    @pl.loop(0, RPT // EK)
    def _wout(k):
      r0 = sid * RPT + k * EK
      pltpu.sync_copy(acc_sh.at[pl.ds(r0, EK)], out_hbm.at[cid, pl.ds(r0, EK)])

  return layer


def _make_score_kernel():
  mesh = plsc.VectorSubcoreMesh(core_axis_name="c", subcore_axis_name="s",
                                num_cores=NC, num_subcores=NS)

  @functools.partial(
      pl.kernel,
      out_type=(
          jax.ShapeDtypeStruct((NB, 16), jnp.float32),
          jax.ShapeDtypeStruct((NB, 16), jnp.float32),
          jax.ShapeDtypeStruct((NB, 16), jnp.float32),
          jax.ShapeDtypeStruct((NB,), jnp.float32),
          jax.ShapeDtypeStruct((NB,), jnp.float32),
          jax.ShapeDtypeStruct((NB,), jnp.float32),
      ),
      mesh=mesh,
      compiler_params=pltpu.CompilerParams(needs_layout_passes=False),
      scratch_types=[
          pltpu.VMEM((BPT,), jnp.int32),
          pltpu.VMEM((BPT,), jnp.int32),
          pltpu.VMEM((BPT,), jnp.int32),
          pltpu.VMEM((BPT,), jnp.int32),
          pltpu.VMEM((BPT, LP), jnp.int32),
          pltpu.VMEM((BPT, ND), jnp.float32),
          pltpu.VMEM((BPT, ND), jnp.float32),
          pltpu.VMEM((BPT, ND), jnp.float32),
          pltpu.VMEM((BPT, ND), jnp.float32),
          pltpu.VMEM((LP, ND), jnp.float32),
          pltpu.VMEM((LP, ND), jnp.float32),
          pltpu.VMEM((LP, ND), jnp.float32),
          pltpu.VMEM((LP, ND), jnp.float32),
          pltpu.VMEM((NI,), jnp.float32),
          pltpu.VMEM((BPT, 16), jnp.float32),
          pltpu.VMEM((BPT, 16), jnp.float32),
          pltpu.VMEM((BPT, 16), jnp.float32),
          pltpu.VMEM((BPT,), jnp.float32),
          pltpu.VMEM((BPT,), jnp.float32),
          pltpu.VMEM((BPT,), jnp.float32),
          pltpu.SemaphoreType.DMA,
          pltpu.SemaphoreType.DMA,
          pltpu.SemaphoreType.DMA,
          pltpu.SemaphoreType.DMA,
          pltpu.SemaphoreType.DMA,
      ],
  )
  def score(items_hbm, user_hbm, bias_hbm, uid_hbm, hist_hbm, pos_hbm,
            neg0_hbm, neg1_hbm, p_hbm, n0_hbm, n1_hbm,
            pb_hbm, n0b_hbm, n1b_hbm,
            uidx_v, pidx_v, n0idx_v, n1idx_v, hidx_v,
            urows_v, prows_v, n0rows_v, n1rows_v,
            hrows0_v, hrows1_v, hrows2_v, hrows3_v,
            bias_v, pbuf_v, n0buf_v, n1buf_v,
            pbias_v, n0bias_v, n1bias_v, bsem,
            hsem0, hsem1, hsem2, hsem3):
    cid = lax.axis_index("c")
    sid = lax.axis_index("s")
    bb = (cid * NS + sid) * BPT

    pltpu.sync_copy(uid_hbm.at[pl.ds(bb, BPT)], uidx_v)
    pltpu.sync_copy(pos_hbm.at[pl.ds(bb, BPT)], pidx_v)
    pltpu.sync_copy(neg0_hbm.at[pl.ds(bb, BPT)], n0idx_v)
    pltpu.sync_copy(neg1_hbm.at[pl.ds(bb, BPT)], n1idx_v)
    pltpu.sync_copy(hist_hbm.at[pl.ds(bb, BPT)], hidx_v)

    hbufs = (hrows0_v, hrows1_v, hrows2_v, hrows3_v)
    hsems = (hsem0, hsem1, hsem2, hsem3)

    def hfire(b, k):
      pltpu.async_copy(items_hbm.at[hidx_v.at[b]], hbufs[k], hsems[k])

    def hdrain(b, k):
      pltpu.make_async_copy(items_hbm.at[hidx_v.at[b]], hbufs[k], hsems[k]).wait()

    pltpu.async_copy(user_hbm.at[uidx_v], urows_v, bsem)
    pltpu.async_copy(items_hbm.at[pidx_v], prows_v, bsem)
    pltpu.async_copy(items_hbm.at[n0idx_v], n0rows_v, bsem)
    pltpu.async_copy(items_hbm.at[n1idx_v], n1rows_v, bsem)
    for k in range(4):
      hfire(k, k)
    pltpu.sync_copy(bias_hbm, bias_v)
    pltpu.make_async_copy(user_hbm.at[uidx_v], urows_v, bsem).wait()
    pltpu.make_async_copy(items_hbm.at[pidx_v], prows_v, bsem).wait()
    pltpu.make_async_copy(items_hbm.at[n0idx_v], n0rows_v, bsem).wait()
    pltpu.make_async_copy(items_hbm.at[n1idx_v], n1rows_v, bsem).wait()

    zero16 = jnp.zeros((16,), jnp.float32)
    inv_l = jnp.float32(1.0 / NL)

    def compute(b, hbuf):
      def hbody(r, accs):
        return tuple(a + hbuf[r, pl.ds(q * 16, 16)]
                     for q, a in enumerate(accs))

      accs = lax.fori_loop(0, NL, hbody, (zero16,) * 8)

      pd = zero16
      n0d = zero16
      n1d = zero16
      for q in range(8):
        sl = pl.ds(q * 16, 16)
        ctx = urows_v[b, sl] + accs[q] * inv_l
        pd = pd + ctx * prows_v[b, sl]
        n0d = n0d + ctx * n0rows_v[b, sl]
        n1d = n1d + ctx * n1rows_v[b, sl]
      pbuf_v[b, pl.ds(0, 16)] = pd
      n0buf_v[b, pl.ds(0, 16)] = n0d
      n1buf_v[b, pl.ds(0, 16)] = n1d

    @pl.loop(0, BPT // 4)
    def _b4(b4):
      for k in range(4):
        b = 4 * b4 + k
        hdrain(b, k)
        compute(b, hbufs[k])

        @pl.when(b + 4 < BPT)
        def _next():
          hfire(b + 4, k)

    for g in range(BPT // 16):
      gsl = pl.ds(g * 16, 16)
      pbias_v[gsl] = plsc.load_gather(bias_v, [pidx_v[gsl]])
      n0bias_v[gsl] = plsc.load_gather(bias_v, [n0idx_v[gsl]])
      n1bias_v[gsl] = plsc.load_gather(bias_v, [n1idx_v[gsl]])

    pltpu.sync_copy(pbuf_v, p_hbm.at[pl.ds(bb, BPT)])
    pltpu.sync_copy(n0buf_v, n0_hbm.at[pl.ds(bb, BPT)])
    pltpu.sync_copy(n1buf_v, n1_hbm.at[pl.ds(bb, BPT)])
    pltpu.sync_copy(pbias_v, pb_hbm.at[pl.ds(bb, BPT)])
    pltpu.sync_copy(n0bias_v, n0b_hbm.at[pl.ds(bb, BPT)])
    pltpu.sync_copy(n1bias_v, n1b_hbm.at[pl.ds(bb, BPT)])

  return score


def _combine_partials(p):
  def body(p_ref, o_ref):
    o_ref[...] = p_ref[0] + p_ref[1]

  return pl.pallas_call(
      body,
      grid=(10,),
      in_specs=[pl.BlockSpec((2, NIP // 10, ND), lambda i: (0, i, 0))],
      out_specs=pl.BlockSpec((NIP // 10, ND), lambda i: (i, 0)),
      out_shape=jax.ShapeDtypeStruct((NIP, ND), jnp.float32),
  )(p)


def _mean_items3(emb0, emb1, p1):
  def body(e0_ref, e1_ref, p_ref, o_ref):
    o_ref[...] = (e0_ref[...] + e1_ref[...] + p_ref[0] + p_ref[1]) * (1.0 / 3.0)

  r = NIP // 10
  return pl.pallas_call(
      body,
      grid=(10,),
      in_specs=[
          pl.BlockSpec((r, ND), lambda i: (i, 0)),
          pl.BlockSpec((r, ND), lambda i: (i, 0)),
          pl.BlockSpec((2, r, ND), lambda i: (0, i, 0)),
      ],
      out_specs=pl.BlockSpec((r, ND), lambda i: (i, 0)),
      out_shape=jax.ShapeDtypeStruct((NIP, ND), jnp.float32),
  )(emb0, emb1, p1)


def _loss_tc(p, n0, n1, pb, n0b, n1b):
  def body(p_ref, n0_ref, n1_ref, pb_ref, n0b_ref, n1b_ref, o_ref):
    ps = jnp.sum(p_ref[...], axis=1, keepdims=True) + pb_ref[...]
    n0s = jnp.sum(n0_ref[...], axis=1, keepdims=True) + n0b_ref[...]
    n1s = jnp.sum(n1_ref[...], axis=1, keepdims=True) + n1b_ref[...]

    def sp(z):
      return jnp.maximum(z, 0.0) + jnp.log1p(jnp.exp(-jnp.abs(z)))

    o_ref[...] = sp(n0s - ps) + sp(n1s - ps)

  return pl.pallas_call(
      body,
      out_shape=jax.ShapeDtypeStruct((NB, 1), jnp.float32),
  )(p, n0, n1, pb.reshape(NB, 1), n0b.reshape(NB, 1), n1b.reshape(NB, 1))


def kernel(user_emb, item_emb, item_bias, edge_vals, edge_index, user_id,
           hist_item_ids, pos_target, replace_candidates, sampled_neg_targets,
           sample_idices):
  del replace_candidates

  pad = E_PAD - NE
  idt = edge_index.dtype
  dst = jnp.concatenate([edge_index[0], jnp.zeros((pad,), idt)])
  dst = dst.astype(jnp.int32).reshape(TOTSUP, SUP, EK)
  src = jnp.concatenate([edge_index[1], jnp.zeros((pad,), idt)])
  src = src.astype(jnp.int32).reshape(TOTSUP, SUP, EK)
  vals = jnp.concatenate([edge_vals, jnp.zeros((pad,), jnp.float32)])
  vals = vals.reshape(TOTSUP, SUP, EK)

  item_emb_pad = jnp.pad(item_emb, ((0, NIP - NI), (0, 0)))
  layer = _make_layer_kernel()
  p0 = layer(item_emb_pad, src, dst, vals)
  emb1 = _combine_partials(p0)
  p1 = layer(emb1, src, dst, vals)
  items = _mean_items3(item_emb_pad, emb1, p1)

  score = _make_score_kernel()
  hist_pad = jnp.pad(hist_item_ids.astype(jnp.int32), ((0, 0), (0, LP - NL)))
  p, n0, n1, pb, n0b, n1b = score(
      items, user_emb, item_bias.reshape(NI),
      user_id.astype(jnp.int32), hist_pad,
      pos_target.astype(jnp.int32),
      sampled_neg_targets[:, 0].astype(jnp.int32),
      sampled_neg_targets[:, 1].astype(jnp.int32),
  )

  loss = _loss_tc(p, n0, n1, pb, n0b, n1b)
  return (loss, sample_idices)


# speedup vs baseline: 6.2156x; 1.0168x over previous
import functools

import jax
import jax.numpy as jnp
from jax import lax
from jax.experimental import pallas as pl
from jax.experimental.pallas import tpu as pltpu
from jax.experimental.pallas import tpu_sc as plsc

NI = 10000
NIP = 10240
ND = 128
NE = 320000
NB = 1024
NL = 50
NC, NS = 2, 16
NW = NC * NS
EK = 128
ECHUNKS = 80
SUP = 8
NSUP = ECHUNKS // SUP
NSUP0 = 17
NSUP1 = 2 * NSUP - NSUP0
ESUP = SUP * EK
EPT = EK * ECHUNKS
E_PAD = NW * EPT
TOTSUP = E_PAD // ESUP
RPT = NIP // NS
LP = 56
BPT = NB // NW


def _make_layer_kernel():
  mesh = plsc.VectorSubcoreMesh(core_axis_name="c", subcore_axis_name="s",
                                num_cores=NC, num_subcores=NS)

  @functools.partial(
      pl.kernel,
      out_type=jax.ShapeDtypeStruct((NC, NIP, ND), jnp.float32),
      mesh=mesh,
      compiler_params=pltpu.CompilerParams(needs_layout_passes=False),
      scratch_types=[
          pltpu.VMEM((2, SUP, EK), jnp.int32),
          pltpu.VMEM((2, SUP, EK), jnp.int32),
          pltpu.VMEM((2, SUP, EK), jnp.float32),
          pltpu.VMEM((EK, ND), jnp.float32),
          pltpu.VMEM((EK, ND), jnp.float32),
          pltpu.VMEM_SHARED((NIP, ND), jnp.float32),
          pltpu.SemaphoreType.DMA,
          pltpu.SemaphoreType.DMA,
          pltpu.SemaphoreType.DMA,
          pltpu.SemaphoreType.DMA,
      ],
  )
  def layer(emb_hbm, src_hbm, dst_hbm, vals_hbm, out_hbm,
            sidx_v, didx_v, vals_v, rows0_v, rows1_v, acc_sh, sem, ssem,
            scsem0, scsem1):
    cid = lax.axis_index("c")
    sid = lax.axis_index("s")
    zero16 = jnp.zeros((16,), jnp.float32)
    nsup = jnp.where(cid == 0, NSUP0, NSUP1)
    bsu = jnp.where(cid == 0, sid * NSUP0, NS * NSUP0 + sid * NSUP1)
    bsu = jnp.minimum(bsu, TOTSUP - 1)

    def stage_fire(sp, sb):
      pltpu.async_copy(src_hbm.at[bsu + sp], sidx_v.at[sb], ssem)
      pltpu.async_copy(dst_hbm.at[bsu + sp], didx_v.at[sb], ssem)
      pltpu.async_copy(vals_hbm.at[bsu + sp], vals_v.at[sb], ssem)

    def stage_drain(sp, sb):
      pltpu.make_async_copy(src_hbm.at[bsu + sp], sidx_v.at[sb], ssem).wait()
      pltpu.make_async_copy(dst_hbm.at[bsu + sp], didx_v.at[sb], ssem).wait()
      pltpu.make_async_copy(vals_hbm.at[bsu + sp], vals_v.at[sb], ssem).wait()

    def fire(sb, c, buf):
      pltpu.async_copy(emb_hbm.at[sidx_v.at[sb, c]], buf, sem)

    def drain(sb, c, buf):
      pltpu.make_async_copy(emb_hbm.at[sidx_v.at[sb, c]], buf, sem).wait()

    def scale(sb, c, buf):
      @pl.loop(0, EK // 16)
      def _scale(g):
        v16 = vals_v[sb, c, pl.ds(g * 16, 16)]
        for e in range(16):
          svec = jnp.broadcast_to(v16[e], (16,))
          row = g * 16 + e
          for q in range(8):
            qsl = pl.ds(q * 16, 16)
            buf[row, qsl] = buf[row, qsl] * svec

    def sfire(sb, c, buf, scsem):
      pltpu.async_copy(buf, acc_sh.at[didx_v.at[sb, c]], scsem, add=True)

    def sdrain(buf, scsem):
      pltpu.make_async_copy(buf, acc_sh.at[didx_v.at[0, 0]], scsem).wait()

    @pl.when(nsup > 0)
    def _prologue():
      stage_fire(0, 0)

    @pl.loop(0, EK)
    def _zrow(i):
      for q in range(8):
        rows1_v[i, pl.ds(q * 16, 16)] = zero16

    @pl.loop(0, RPT // EK)
    def _zacc(k):
      pltpu.sync_copy(rows1_v, acc_sh.at[pl.ds(sid * RPT + k * EK, EK)])

    plsc.subcore_barrier()

    @pl.loop(0, nsup)
    def _sup(sp):
      sb = sp % 2
      stage_drain(sp, sb)

      @pl.when(sp + 1 < nsup)
      def _nextstage():
        stage_fire(sp + 1, 1 - sb)

      fire(sb, 0, rows0_v)

      @pl.loop(0, SUP // 2)
      def _c2(c2):
        c0 = 2 * c2
        c1 = c0 + 1
        drain(sb, c0, rows0_v)
        scale(sb, c0, rows0_v)

        @pl.when((sp > 0) | (c2 > 0))
        def _sd1():
          sdrain(rows1_v, scsem1)

        fire(sb, c1, rows1_v)
        sfire(sb, c0, rows0_v, scsem0)
        drain(sb, c1, rows1_v)
        scale(sb, c1, rows1_v)
        sdrain(rows0_v, scsem0)

        @pl.when(c1 + 1 < SUP)
        def _nextrow():
          fire(sb, c1 + 1, rows0_v)

        sfire(sb, c1, rows1_v, scsem1)

    @pl.when(nsup > 0)
    def _finaldrain():
      sdrain(rows1_v, scsem1)

    plsc.subcore_barrier()

    @pl.loop(0, RPT // EK)
    def _wout(k):
      r0 = sid * RPT + k * EK
      pltpu.sync_copy(acc_sh.at[pl.ds(r0, EK)], out_hbm.at[cid, pl.ds(r0, EK)])

  return layer


def _make_score_kernel():
  mesh = plsc.VectorSubcoreMesh(core_axis_name="c", subcore_axis_name="s",
                                num_cores=NC, num_subcores=NS)

  @functools.partial(
      pl.kernel,
      out_type=(
          jax.ShapeDtypeStruct((NB, 16), jnp.float32),
          jax.ShapeDtypeStruct((NB, 16), jnp.float32),
          jax.ShapeDtypeStruct((NB, 16), jnp.float32),
          jax.ShapeDtypeStruct((NB,), jnp.float32),
          jax.ShapeDtypeStruct((NB,), jnp.float32),
          jax.ShapeDtypeStruct((NB,), jnp.float32),
      ),
      mesh=mesh,
      compiler_params=pltpu.CompilerParams(needs_layout_passes=False),
      scratch_types=[
          pltpu.VMEM((BPT,), jnp.int32),
          pltpu.VMEM((BPT,), jnp.int32),
          pltpu.VMEM((BPT,), jnp.int32),
          pltpu.VMEM((BPT,), jnp.int32),
          pltpu.VMEM((BPT, LP), jnp.int32),
          pltpu.VMEM((BPT, ND), jnp.float32),
          pltpu.VMEM((BPT, ND), jnp.float32),
          pltpu.VMEM((BPT, ND), jnp.float32),
          pltpu.VMEM((BPT, ND), jnp.float32),
          pltpu.VMEM((LP, ND), jnp.float32),
          pltpu.VMEM((LP, ND), jnp.float32),
          pltpu.VMEM((LP, ND), jnp.float32),
          pltpu.VMEM((LP, ND), jnp.float32),
          pltpu.VMEM((NI,), jnp.float32),
          pltpu.VMEM((BPT, 16), jnp.float32),
          pltpu.VMEM((BPT, 16), jnp.float32),
          pltpu.VMEM((BPT, 16), jnp.float32),
          pltpu.VMEM((BPT,), jnp.float32),
          pltpu.VMEM((BPT,), jnp.float32),
          pltpu.VMEM((BPT,), jnp.float32),
          pltpu.SemaphoreType.DMA,
          pltpu.SemaphoreType.DMA,
          pltpu.SemaphoreType.DMA,
          pltpu.SemaphoreType.DMA,
          pltpu.SemaphoreType.DMA,
      ],
  )
  def score(items_hbm, user_hbm, bias_hbm, uid_hbm, hist_hbm, pos_hbm,
            neg0_hbm, neg1_hbm, p_hbm, n0_hbm, n1_hbm,
            pb_hbm, n0b_hbm, n1b_hbm,
            uidx_v, pidx_v, n0idx_v, n1idx_v, hidx_v,
            urows_v, prows_v, n0rows_v, n1rows_v,
            hrows0_v, hrows1_v, hrows2_v, hrows3_v,
            bias_v, pbuf_v, n0buf_v, n1buf_v,
            pbias_v, n0bias_v, n1bias_v, bsem,
            hsem0, hsem1, hsem2, hsem3):
    cid = lax.axis_index("c")
    sid = lax.axis_index("s")
    bb = (cid * NS + sid) * BPT

    pltpu.sync_copy(uid_hbm.at[pl.ds(bb, BPT)], uidx_v)
    pltpu.sync_copy(pos_hbm.at[pl.ds(bb, BPT)], pidx_v)
    pltpu.sync_copy(neg0_hbm.at[pl.ds(bb, BPT)], n0idx_v)
    pltpu.sync_copy(neg1_hbm.at[pl.ds(bb, BPT)], n1idx_v)
    pltpu.sync_copy(hist_hbm.at[pl.ds(bb, BPT)], hidx_v)

    hbufs = (hrows0_v, hrows1_v, hrows2_v, hrows3_v)
    hsems = (hsem0, hsem1, hsem2, hsem3)

    def hfire(b, k):
      pltpu.async_copy(items_hbm.at[hidx_v.at[b]], hbufs[k], hsems[k])

    def hdrain(b, k):
      pltpu.make_async_copy(items_hbm.at[hidx_v.at[b]], hbufs[k], hsems[k]).wait()

    pltpu.async_copy(user_hbm.at[uidx_v], urows_v, bsem)
    pltpu.async_copy(items_hbm.at[pidx_v], prows_v, bsem)
    pltpu.async_copy(items_hbm.at[n0idx_v], n0rows_v, bsem)
    pltpu.async_copy(items_hbm.at[n1idx_v], n1rows_v, bsem)
    for k in range(4):
      hfire(k, k)
    pltpu.sync_copy(bias_hbm, bias_v)
    pltpu.make_async_copy(user_hbm.at[uidx_v], urows_v, bsem).wait()
    pltpu.make_async_copy(items_hbm.at[pidx_v], prows_v, bsem).wait()
    pltpu.make_async_copy(items_hbm.at[n0idx_v], n0rows_v, bsem).wait()
    pltpu.make_async_copy(items_hbm.at[n1idx_v], n1rows_v, bsem).wait()

    zero16 = jnp.zeros((16,), jnp.float32)
    inv_l = jnp.float32(1.0 / NL)

    def compute(b, hbuf):
      def hbody(r, accs):
        return tuple(a + hbuf[r, pl.ds(q * 16, 16)]
                     for q, a in enumerate(accs))

      accs = lax.fori_loop(0, NL, hbody, (zero16,) * 8)

      pd = zero16
      n0d = zero16
      n1d = zero16
      for q in range(8):
        sl = pl.ds(q * 16, 16)
        ctx = urows_v[b, sl] + accs[q] * inv_l
        pd = pd + ctx * prows_v[b, sl]
        n0d = n0d + ctx * n0rows_v[b, sl]
        n1d = n1d + ctx * n1rows_v[b, sl]
      pbuf_v[b, pl.ds(0, 16)] = pd
      n0buf_v[b, pl.ds(0, 16)] = n0d
      n1buf_v[b, pl.ds(0, 16)] = n1d

    @pl.loop(0, BPT // 4)
    def _b4(b4):
      for k in range(4):
        b = 4 * b4 + k
        hdrain(b, k)
        compute(b, hbufs[k])

        @pl.when(b + 4 < BPT)
        def _next():
          hfire(b + 4, k)

    for g in range(BPT // 16):
      gsl = pl.ds(g * 16, 16)
      pbias_v[gsl] = plsc.load_gather(bias_v, [pidx_v[gsl]])
      n0bias_v[gsl] = plsc.load_gather(bias_v, [n0idx_v[gsl]])
      n1bias_v[gsl] = plsc.load_gather(bias_v, [n1idx_v[gsl]])

    pltpu.sync_copy(pbuf_v, p_hbm.at[pl.ds(bb, BPT)])
    pltpu.sync_copy(n0buf_v, n0_hbm.at[pl.ds(bb, BPT)])
    pltpu.sync_copy(n1buf_v, n1_hbm.at[pl.ds(bb, BPT)])
    pltpu.sync_copy(pbias_v, pb_hbm.at[pl.ds(bb, BPT)])
    pltpu.sync_copy(n0bias_v, n0b_hbm.at[pl.ds(bb, BPT)])
    pltpu.sync_copy(n1bias_v, n1b_hbm.at[pl.ds(bb, BPT)])

  return score


def _combine_partials(p):
  def body(p_ref, o_ref):
    o_ref[...] = p_ref[0] + p_ref[1]

  return pl.pallas_call(
      body,
      grid=(10,),
      in_specs=[pl.BlockSpec((2, NIP // 10, ND), lambda i: (0, i, 0))],
      out_specs=pl.BlockSpec((NIP // 10, ND), lambda i: (i, 0)),
      out_shape=jax.ShapeDtypeStruct((NIP, ND), jnp.float32),
  )(p)


def _mean_items3(emb0, emb1, p1):
  def body(e0_ref, e1_ref, p_ref, o_ref):
    o_ref[...] = (e0_ref[...] + e1_ref[...] + p_ref[0] + p_ref[1]) * (1.0 / 3.0)

  r = NIP // 10
  return pl.pallas_call(
      body,
      grid=(10,),
      in_specs=[
          pl.BlockSpec((r, ND), lambda i: (i, 0)),
          pl.BlockSpec((r, ND), lambda i: (i, 0)),
          pl.BlockSpec((2, r, ND), lambda i: (0, i, 0)),
      ],
      out_specs=pl.BlockSpec((r, ND), lambda i: (i, 0)),
      out_shape=jax.ShapeDtypeStruct((NIP, ND), jnp.float32),
  )(emb0, emb1, p1)


def _loss_tc(p, n0, n1, pb, n0b, n1b):
  def body(p_ref, n0_ref, n1_ref, pb_ref, n0b_ref, n1b_ref, o_ref):
    ps = jnp.sum(p_ref[...], axis=1, keepdims=True) + pb_ref[...]
    n0s = jnp.sum(n0_ref[...], axis=1, keepdims=True) + n0b_ref[...]
    n1s = jnp.sum(n1_ref[...], axis=1, keepdims=True) + n1b_ref[...]

    def sp(z):
      return jnp.maximum(z, 0.0) + jnp.log1p(jnp.exp(-jnp.abs(z)))

    o_ref[...] = sp(n0s - ps) + sp(n1s - ps)

  return pl.pallas_call(
      body,
      out_shape=jax.ShapeDtypeStruct((NB, 1), jnp.float32),
  )(p, n0, n1, pb.reshape(NB, 1), n0b.reshape(NB, 1), n1b.reshape(NB, 1))


def kernel(user_emb, item_emb, item_bias, edge_vals, edge_index, user_id,
           hist_item_ids, pos_target, replace_candidates, sampled_neg_targets,
           sample_idices):
  del replace_candidates

  pad = E_PAD - NE
  idt = edge_index.dtype
  dst = jnp.concatenate([edge_index[0], jnp.zeros((pad,), idt)])
  dst = dst.astype(jnp.int32).reshape(TOTSUP, SUP, EK)
  src = jnp.concatenate([edge_index[1], jnp.zeros((pad,), idt)])
  src = src.astype(jnp.int32).reshape(TOTSUP, SUP, EK)
  vals = jnp.concatenate([edge_vals, jnp.zeros((pad,), jnp.float32)])
  vals = vals.reshape(TOTSUP, SUP, EK)

  item_emb_pad = jnp.pad(item_emb, ((0, NIP - NI), (0, 0)))
  layer = _make_layer_kernel()
  p0 = layer(item_emb_pad, src, dst, vals)
  emb1 = _combine_partials(p0)
  p1 = layer(emb1, src, dst, vals)
  items = _mean_items3(item_emb_pad, emb1, p1)

  score = _make_score_kernel()
  hist_pad = jnp.pad(hist_item_ids.astype(jnp.int32), ((0, 0), (0, LP - NL)))
  p, n0, n1, pb, n0b, n1b = score(
      items, user_emb, item_bias.reshape(NI),
      user_id.astype(jnp.int32), hist_pad,
      pos_target.astype(jnp.int32),
      sampled_neg_targets[:, 0].astype(jnp.int32),
      sampled_neg_targets[:, 1].astype(jnp.int32),
  )

  loss = _loss_tc(p, n0, n1, pb, n0b, n1b)
  return (loss, sample_idices)
